# baseline (speedup 1.0000x reference)
import jax
import jax.numpy as jnp
from jax import lax
from jax.experimental import pallas as pl
from jax.experimental.pallas import tpu as pltpu
from jax.experimental.pallas import tpu_sc as plsc

NUM_CORES = 2
NUM_SUBCORES = 16
NUM_WORKERS = NUM_CORES * NUM_SUBCORES
LANES = 16

BATCH = 16384
K = 64
N_PER = BATCH // NUM_WORKERS
CHUNK = 128
N_CHUNKS = N_PER // CHUNK
GROUPS = CHUNK // LANES
PAIR = 2 * K


def _body(x_hbm, s_hbm, r_hbm, w_hbm, out_hbm,
          x_stage, idx_v, par_v, s_buf, r_buf, w_buf, out_v, sem):
  wid = lax.axis_index("s") * NUM_CORES + lax.axis_index("c")
  base = wid * N_PER

  lane = lax.iota(jnp.int32, LANES)

  pltpu.sync_copy(x_hbm.at[pl.ds(base * 3, N_PER * 3)], x_stage)
  for t in range(3):
    for j in range(N_CHUNKS):
      for i in range(CHUNK // LANES):
        src = (j * CHUNK + i * LANES + lane) * 3 + t
        orig = plsc.load_gather(x_stage, [src])
        row = t * N_CHUNKS + j
        idx_v[row, pl.ds(i * LANES, LANES)] = orig >> 1
        par_v[row, pl.ds(i * LANES, LANES)] = (orig & 1) * K

  for j in range(N_CHUNKS):
    cs = pltpu.async_copy(s_hbm.at[idx_v.at[0 * N_CHUNKS + j]], s_buf, sem)
    cr = pltpu.async_copy(r_hbm.at[idx_v.at[1 * N_CHUNKS + j]], r_buf, sem)
    cw = pltpu.async_copy(w_hbm.at[idx_v.at[2 * N_CHUNKS + j]], w_buf, sem)
    cs.wait()
    cr.wait()
    cw.wait()

    def group_body(g, _, j=j):
      rows = g * LANES + lane
      par_s = par_v[0 * N_CHUNKS + j, pl.ds(g * LANES, LANES)]
      par_r = par_v[1 * N_CHUNKS + j, pl.ds(g * LANES, LANES)]
      par_w = par_v[2 * N_CHUNKS + j, pl.ds(g * LANES, LANES)]
      acc = jnp.zeros((LANES,), jnp.float32)
      for k in range(K):
        col = (lane + k) & (K - 1)
        sv = plsc.load_gather(s_buf, [rows, par_s + col])
        rv = plsc.load_gather(r_buf, [rows, par_r + col])
        wv = plsc.load_gather(w_buf, [rows, par_w + col])
        acc = acc + (sv + rv) * wv
      out_v[pl.ds(j * CHUNK + g * LANES, LANES)] = (
          1.0 / (1.0 + jnp.exp(-acc)))
      return 0

    lax.fori_loop(0, GROUPS, group_body, 0)

  pltpu.sync_copy(out_v, out_hbm.at[pl.ds(base, N_PER)])


@jax.jit
def kernel(X, s_table, r_table, w_table):
  mesh = plsc.VectorSubcoreMesh(core_axis_name="c", subcore_axis_name="s")
  run = pl.kernel(
      _body,
      out_type=jax.ShapeDtypeStruct((BATCH,), jnp.float32),
      mesh=mesh,
      scratch_types=[
          pltpu.VMEM((N_PER * 3,), jnp.int32),
          pltpu.VMEM((3 * N_CHUNKS, CHUNK), jnp.int32),
          pltpu.VMEM((3 * N_CHUNKS, CHUNK), jnp.int32),
          pltpu.VMEM((CHUNK, PAIR), jnp.float32),
          pltpu.VMEM((CHUNK, PAIR), jnp.float32),
          pltpu.VMEM((CHUNK, PAIR), jnp.float32),
          pltpu.VMEM((N_PER,), jnp.float32),
          pltpu.SemaphoreType.DMA,
      ],
      compiler_params=pltpu.CompilerParams(needs_layout_passes=False),
  )
  def pair_rows(t):
    return jnp.concatenate([t[0::2], t[1::2]], axis=1)

  return run(X.reshape(-1),
             pair_rows(s_table),
             pair_rows(r_table),
             pair_rows(w_table))


# speedup vs baseline: 10.3281x; 10.3281x over previous
import jax
import jax.numpy as jnp
from jax import lax
from jax.experimental import pallas as pl
from jax.experimental.pallas import tpu as pltpu
from jax.experimental.pallas import tpu_sc as plsc

NUM_CORES = 2
NUM_SUBCORES = 16
NUM_WORKERS = NUM_CORES * NUM_SUBCORES
LANES = 16

BATCH = 16384
K = 64
V_CNT = 100000
N_SLABS = 3 * K
SLABS_PER_W = N_SLABS // NUM_WORKERS
HALF_A = 50048
HALF_B = V_CNT - HALF_A
ECHUNK = 2048
N_ECHUNKS = BATCH // ECHUNK
N_PER = BATCH // NUM_WORKERS


def _extract_body(xs_hbm, xr_hbm, xw_hbm, s_hbm, r_hbm, w_hbm, v_hbm,
                  half_a, half_b, idx_c, out_b, sem_a, sem_b, sem_o):
  wid = lax.axis_index("s") * NUM_CORES + lax.axis_index("c")
  zero16 = jnp.zeros((LANES,), jnp.int32)
  x_tabs = (xs_hbm, xr_hbm, xw_hbm)
  tabs = (s_hbm, r_hbm, w_hbm)

  def slab_k(i):
    return (i * NUM_WORKERS + wid) % K

  def slab_dma(i, half, dst, sem):
    k_rt = slab_k(i)
    off = 0 if half == 0 else HALF_A
    n = HALF_A if half == 0 else HALF_B
    return pltpu.async_copy(
        tabs[i // 2].at[pl.ds(k_rt, 1), pl.ds(off, n)], dst, sem)

  def extract_pass(i, half, first):
    off = 0 if half == 0 else HALF_A
    n = HALF_A if half == 0 else HALF_B
    buf = half_a if half == 0 else half_b
    x_hbm = x_tabs[i // 2]

    def chunk_body(c, _):
      pltpu.sync_copy(x_hbm.at[pl.ds(c * ECHUNK, ECHUNK)], idx_c)

      def group_body(g, _, c=c):
        iv = idx_c[pl.ds(g * LANES, LANES)]
        loc = iv - off
        m = (loc >= 0) & (loc < n)
        vals = plsc.load_gather(buf, [zero16, jnp.where(m, loc, 0)],
                                mask=m)
        vals = jnp.where(m, vals, 0.0)
        e = c * ECHUNK + g * LANES
        if first:
          out_b[0, pl.ds(e, LANES)] = vals
        else:
          out_b[0, pl.ds(e, LANES)] = out_b[0, pl.ds(e, LANES)] + vals
        return 0

      lax.fori_loop(0, ECHUNK // LANES, group_body, 0)
      return 0

    lax.fori_loop(0, N_ECHUNKS, chunk_body, 0)

  for i in range(SLABS_PER_W):
    if i == 0:
      slab_dma(0, 0, half_a, sem_a).wait()
    cb = slab_dma(i, 1, half_b, sem_b)
    extract_pass(i, 0, first=True)
    cb.wait()
    ca = None
    if i + 1 < SLABS_PER_W:
      ca = slab_dma(i + 1, 0, half_a, sem_a)
    extract_pass(i, 1, first=False)
    row = i * NUM_WORKERS + wid
    pltpu.async_copy(out_b, v_hbm.at[pl.ds(row, 1), pl.ds(0, BATCH)],
                     sem_o).wait()
    if ca is not None:
      ca.wait()


def _reduce_body(v_hbm, out_hbm, v_buf, out_v, sem):
  wid = lax.axis_index("s") * NUM_CORES + lax.axis_index("c")
  base = wid * N_PER
  pltpu.async_copy(v_hbm.at[pl.ds(0, N_SLABS), pl.ds(base, N_PER)],
                   v_buf, sem).wait()

  def group_body(g, _):
    acc = jnp.zeros((LANES,), jnp.float32)

    def k_body(k, acc):
      sv = v_buf[0 * K + k, pl.ds(g * LANES, LANES)]
      rv = v_buf[1 * K + k, pl.ds(g * LANES, LANES)]
      wv = v_buf[2 * K + k, pl.ds(g * LANES, LANES)]
      return acc + (sv + rv) * wv

    acc = lax.fori_loop(0, K, k_body, acc)
    out_v[pl.ds(g * LANES, LANES)] = 1.0 / (1.0 + jnp.exp(-acc))
    return 0

  lax.fori_loop(0, N_PER // LANES, group_body, 0)
  pltpu.sync_copy(out_v, out_hbm.at[pl.ds(base, N_PER)])


@jax.jit
def kernel(X, s_table, r_table, w_table):
  mesh = plsc.VectorSubcoreMesh(core_axis_name="c", subcore_axis_name="s")
  extract = pl.kernel(
      _extract_body,
      out_type=jax.ShapeDtypeStruct((N_SLABS, BATCH), jnp.float32),
      mesh=mesh,
      scratch_types=[
          pltpu.VMEM((1, HALF_A), jnp.float32),
          pltpu.VMEM((1, HALF_B), jnp.float32),
          pltpu.VMEM((ECHUNK,), jnp.int32),
          pltpu.VMEM((1, BATCH), jnp.float32),
          pltpu.SemaphoreType.DMA,
          pltpu.SemaphoreType.DMA,
          pltpu.SemaphoreType.DMA,
      ],
      compiler_params=pltpu.CompilerParams(needs_layout_passes=False),
  )
  reduce = pl.kernel(
      _reduce_body,
      out_type=jax.ShapeDtypeStruct((BATCH,), jnp.float32),
      mesh=mesh,
      scratch_types=[
          pltpu.VMEM((N_SLABS, N_PER), jnp.float32),
          pltpu.VMEM((N_PER,), jnp.float32),
          pltpu.SemaphoreType.DMA,
      ],
      compiler_params=pltpu.CompilerParams(needs_layout_passes=False),
  )
  v = extract(X[:, 0], X[:, 1], X[:, 2],
              s_table.T, r_table.T, w_table.T)
  return reduce(v)


# speedup vs baseline: 24.8952x; 2.4104x over previous
import jax
import jax.numpy as jnp
from jax import lax
from jax.experimental import pallas as pl
from jax.experimental.pallas import tpu as pltpu
from jax.experimental.pallas import tpu_sc as plsc

NUM_CORES = 2
NUM_SUBCORES = 16
NUM_WORKERS = NUM_CORES * NUM_SUBCORES
LANES = 16

BATCH = 16384
K = 64
V_CNT = 100000
N_SLABS = 3 * K
SLABS_PER_W = N_SLABS // NUM_WORKERS
HALF_A = 50048
HALF_B = V_CNT - HALF_A
ECHUNK = 2048
N_ECHUNKS = BATCH // ECHUNK
N_PER = BATCH // NUM_WORKERS


def _extract_body(xs_hbm, xr_hbm, xw_hbm, s_hbm, r_hbm, w_hbm, v_hbm,
                  slab_b, idx_p, idx_q, out_b, sem_s, sem_p, sem_q, sem_o):
  wid = lax.axis_index("s") * NUM_CORES + lax.axis_index("c")
  zero16 = jnp.zeros((LANES,), jnp.int32)
  x_tabs = (xs_hbm, xr_hbm, xw_hbm)
  tabs = (s_hbm, r_hbm, w_hbm)
  idx_bufs = (idx_p, idx_q)
  idx_sems = (sem_p, sem_q)

  co = None
  for i in range(SLABS_PER_W):
    k_rt = (i * NUM_WORKERS + wid) % K
    cs = pltpu.async_copy(
        tabs[i // 2].at[pl.ds(k_rt, 1), pl.ds(0, V_CNT)], slab_b, sem_s)
    x_hbm = x_tabs[i // 2]
    ci = pltpu.async_copy(x_hbm.at[pl.ds(0, ECHUNK)], idx_bufs[0],
                          idx_sems[0])
    cs.wait()
    if co is not None:
      co.wait()

    for c in range(N_ECHUNKS):
      ci.wait()
      if c + 1 < N_ECHUNKS:
        nb = (c + 1) % 2
        ci = pltpu.async_copy(
            x_hbm.at[pl.ds((c + 1) * ECHUNK, ECHUNK)], idx_bufs[nb],
            idx_sems[nb])
      ib = idx_bufs[c % 2]

      def chunk_part(u, _, c=c, ib=ib):
        for g8 in range(8):
          g = u * 8 + g8
          iv = ib[pl.ds(g * LANES, LANES)]
          vals = plsc.load_gather(slab_b, [zero16, iv])
          out_b[0, pl.ds(c * ECHUNK + g * LANES, LANES)] = vals
        return 0

      lax.fori_loop(0, ECHUNK // LANES // 8, chunk_part, 0)

    row = i * NUM_WORKERS + wid
    co = pltpu.async_copy(out_b, v_hbm.at[pl.ds(row, 1), pl.ds(0, BATCH)],
                          sem_o)
  co.wait()


def _reduce_body(v_hbm, out_hbm, v_buf, out_v, sem):
  wid = lax.axis_index("s") * NUM_CORES + lax.axis_index("c")
  base = wid * N_PER
  pltpu.async_copy(v_hbm.at[pl.ds(0, N_SLABS), pl.ds(base, N_PER)],
                   v_buf, sem).wait()

  def group_body(g, _):
    acc = jnp.zeros((LANES,), jnp.float32)

    def k_body(k8, acc):
      for kk in range(8):
        k = k8 * 8 + kk
        sv = v_buf[0 * K + k, pl.ds(g * LANES, LANES)]
        rv = v_buf[1 * K + k, pl.ds(g * LANES, LANES)]
        wv = v_buf[2 * K + k, pl.ds(g * LANES, LANES)]
        acc = acc + (sv + rv) * wv
      return acc

    acc = lax.fori_loop(0, K // 8, k_body, acc)
    out_v[pl.ds(g * LANES, LANES)] = 1.0 / (1.0 + jnp.exp(-acc))
    return 0

  lax.fori_loop(0, N_PER // LANES, group_body, 0)
  pltpu.sync_copy(out_v, out_hbm.at[pl.ds(base, N_PER)])


@jax.jit
def kernel(X, s_table, r_table, w_table):
  mesh = plsc.VectorSubcoreMesh(core_axis_name="c", subcore_axis_name="s")
  extract = pl.kernel(
      _extract_body,
      out_type=jax.ShapeDtypeStruct((N_SLABS, BATCH), jnp.float32),
      mesh=mesh,
      scratch_types=[
          pltpu.VMEM((1, V_CNT), jnp.float32),
          pltpu.VMEM((ECHUNK,), jnp.int32),
          pltpu.VMEM((ECHUNK,), jnp.int32),
          pltpu.VMEM((1, BATCH), jnp.float32),
          pltpu.SemaphoreType.DMA,
          pltpu.SemaphoreType.DMA,
          pltpu.SemaphoreType.DMA,
          pltpu.SemaphoreType.DMA,
      ],
      compiler_params=pltpu.CompilerParams(needs_layout_passes=False),
  )
  reduce = pl.kernel(
      _reduce_body,
      out_type=jax.ShapeDtypeStruct((BATCH,), jnp.float32),
      mesh=mesh,
      scratch_types=[
          pltpu.VMEM((N_SLABS, N_PER), jnp.float32),
          pltpu.VMEM((N_PER,), jnp.float32),
          pltpu.SemaphoreType.DMA,
      ],
      compiler_params=pltpu.CompilerParams(needs_layout_passes=False),
  )
  v = extract(X[:, 0], X[:, 1], X[:, 2],
              s_table.T, r_table.T, w_table.T)
  return reduce(v)
